# baseline (device time: 339303 ns/iter reference)
import jax
import jax.numpy as jnp
from jax import lax
from jax.experimental import pallas as pl
from jax.experimental.pallas import tpu as pltpu

N_DEV = 4
NBS = 256


def kernel(x, w_mat):
    m_per, k = x.shape
    k2, n_per = w_mat.shape
    assert k == k2
    half = m_per // 2
    quart = half // 2
    n_tiles = n_per // NBS

    def body(x_ref, w_ref, out_ref,
             local_ref, left_ref, right_ref, w_stream, y_vmem,
             send_a, recv_a, send_b, recv_b,
             ws_sems, y_sems, stage_sem):
        my_pos = lax.axis_index("i")
        left = (my_pos - 1) % N_DEV
        right = (my_pos + 1) % N_DEV

        barrier_sem = pltpu.get_barrier_semaphore()
        for nbr in [left, right]:
            pl.semaphore_signal(
                barrier_sem, inc=1,
                device_id=(nbr,), device_id_type=pl.DeviceIdType.MESH,
            )
        pl.semaphore_wait(barrier_sem, 2)

        pending = {}

        def drain(d):
            cp = pending.pop(d, None)
            if cp is not None:
                cp.wait()

        def compute_pair(ref_a, ra0, base_a, ref_b, rb0, base_b, nr):
            def emit(d, vals, base, col0):
                drain(d)
                y_vmem[d, pl.ds(0, nr), :] = jnp.maximum(vals, 0.0)
                cp = pltpu.make_async_copy(
                    y_vmem.at[d, pl.ds(0, nr), :],
                    out_ref.at[pl.ds(base, nr), pl.ds(col0, NBS)],
                    y_sems.at[d])
                cp.start()
                pending[d] = cp

            scps = [
                pltpu.make_async_copy(
                    w_ref.at[:, pl.ds(s * NBS, NBS)],
                    w_stream.at[s % 2], ws_sems.at[s % 2])
                for s in range(n_tiles)
            ]
            scps[0].start()
            scps[1].start()
            for s in range(n_tiles):
                scps[s].wait()
                ya = jnp.dot(ref_a[pl.ds(ra0, nr), :], w_stream[s % 2],
                             preferred_element_type=jnp.float32)
                emit(0, ya, base_a, s * NBS)
                yb = jnp.dot(ref_b[pl.ds(rb0, nr), :], w_stream[s % 2],
                             preferred_element_type=jnp.float32)
                emit(1, yb, base_b, s * NBS)
                if s + 2 < n_tiles:
                    scps[s + 2].start()

        rdma_a_r = pltpu.make_async_remote_copy(
            src_ref=x_ref, dst_ref=left_ref,
            send_sem=send_a.at[0], recv_sem=recv_a.at[0],
            device_id=(right,), device_id_type=pl.DeviceIdType.MESH)
        rdma_a_l = pltpu.make_async_remote_copy(
            src_ref=x_ref, dst_ref=right_ref,
            send_sem=send_a.at[1], recv_sem=recv_a.at[1],
            device_id=(left,), device_id_type=pl.DeviceIdType.MESH)
        rdma_a_r.start()
        rdma_a_l.start()

        st = pltpu.make_async_copy(x_ref, local_ref, stage_sem)
        st.start()
        st.wait()
        compute_pair(local_ref, 0, my_pos * m_per,
                     local_ref, half, my_pos * m_per + half, half)

        rdma_a_r.wait_recv()
        fwd_r = []
        for q in range(2):
            r = pltpu.make_async_remote_copy(
                src_ref=left_ref.at[pl.ds(q * quart, quart), :],
                dst_ref=local_ref.at[pl.ds(q * quart, quart), :],
                send_sem=send_b.at[q], recv_sem=recv_b.at[q],
                device_id=(right,), device_id_type=pl.DeviceIdType.MESH)
            r.start()
            fwd_r.append(r)
        rdma_a_l.wait_recv()
        fwd_l = []
        for q in range(2):
            r = pltpu.make_async_remote_copy(
                src_ref=right_ref.at[pl.ds(half + q * quart, quart), :],
                dst_ref=local_ref.at[pl.ds(half + q * quart, quart), :],
                send_sem=send_b.at[2 + q], recv_sem=recv_b.at[2 + q],
                device_id=(left,), device_id_type=pl.DeviceIdType.MESH)
            r.start()
            fwd_l.append(r)

        compute_pair(left_ref, 0, left * m_per,
                     right_ref, 0, right * m_per, half)
        compute_pair(left_ref, half, left * m_per + half,
                     right_ref, half, right * m_per + half, half)
        rdma_a_r.wait_send()
        rdma_a_l.wait_send()

        diag = (my_pos + 2) % N_DEV
        fwd_r[0].wait_recv()
        fwd_l[0].wait_recv()
        compute_pair(local_ref, 0, diag * m_per,
                     local_ref, half, diag * m_per + half, quart)
        fwd_r[1].wait_recv()
        fwd_l[1].wait_recv()
        for r in fwd_r + fwd_l:
            r.wait_send()
        compute_pair(local_ref, quart, diag * m_per + quart,
                     local_ref, half + quart, diag * m_per + half + quart,
                     quart)

        drain(0)
        drain(1)

    return pl.pallas_call(
        body,
        out_shape=jax.ShapeDtypeStruct((N_DEV * m_per, n_per), jnp.float32),
        in_specs=[
            pl.BlockSpec(memory_space=pl.ANY),
            pl.BlockSpec(memory_space=pl.ANY),
        ],
        out_specs=pl.BlockSpec(memory_space=pl.ANY),
        scratch_shapes=[
            pltpu.VMEM((m_per, k), jnp.float32),
            pltpu.VMEM((m_per, k), jnp.float32),
            pltpu.VMEM((m_per, k), jnp.float32),
            pltpu.VMEM((2, k, NBS), jnp.float32),
            pltpu.VMEM((2, half, NBS), jnp.float32),
            pltpu.SemaphoreType.DMA((2,)),
            pltpu.SemaphoreType.DMA((2,)),
            pltpu.SemaphoreType.DMA((4,)),
            pltpu.SemaphoreType.DMA((4,)),
            pltpu.SemaphoreType.DMA((2,)),
            pltpu.SemaphoreType.DMA((2,)),
            pltpu.SemaphoreType.DMA,
        ],
        compiler_params=pltpu.CompilerParams(
            collective_id=0, vmem_limit_bytes=64 * 1024 * 1024),
    )(x, w_mat)


# device time: 189173 ns/iter; 1.7936x vs baseline; 1.7936x over previous
import jax
import jax.numpy as jnp
from jax import lax
from jax.experimental import pallas as pl
from jax.experimental.pallas import tpu as pltpu

N_DEV = 4
XR = 128
WC = 128
NBC = 512


def kernel(x, w_mat):
    m_per, k = x.shape
    k2, n_per = w_mat.shape
    assert k == k2
    half = m_per // 2
    quart = half // 2
    n_tiles = n_per // NBC

    def body(x_ref, w_ref, out_ref,
             local_bf, left_bf, right_bf, diag_bf, w_bf,
             xf32, wf32, y_vmem,
             send_a, recv_a, send_b, recv_b,
             xc_sems, wc_sems, y_sems):
        my_pos = lax.axis_index("i")
        left = (my_pos - 1) % N_DEV
        right = (my_pos + 1) % N_DEV

        barrier_sem = pltpu.get_barrier_semaphore()
        for nbr in [left, right]:
            pl.semaphore_signal(
                barrier_sem, inc=1,
                device_id=(nbr,), device_id_type=pl.DeviceIdType.MESH,
            )
        pl.semaphore_wait(barrier_sem, 2)

        pending = {}

        def drain(d):
            cp = pending.pop(d, None)
            if cp is not None:
                cp.wait()

        n_xt = m_per // XR
        for t in range(2):
            pltpu.make_async_copy(
                x_ref.at[pl.ds(t * XR, XR), :], xf32.at[t],
                xc_sems.at[t]).start()

        def xconv(t, carry):
            s = t % 2
            pltpu.make_async_copy(
                x_ref.at[pl.ds(t * XR, XR), :], xf32.at[s],
                xc_sems.at[s]).wait()
            local_bf[pl.ds(t * XR, XR), :] = xf32[s].astype(jnp.bfloat16)

            @pl.when(t + 2 < n_xt)
            def _():
                pltpu.make_async_copy(
                    x_ref.at[pl.ds((t + 2) * XR, XR), :], xf32.at[s],
                    xc_sems.at[s]).start()
            return carry

        lax.fori_loop(0, n_xt, xconv, 0)

        rdma_a_r = pltpu.make_async_remote_copy(
            src_ref=local_bf, dst_ref=left_bf,
            send_sem=send_a.at[0], recv_sem=recv_a.at[0],
            device_id=(right,), device_id_type=pl.DeviceIdType.MESH)
        rdma_a_l = pltpu.make_async_remote_copy(
            src_ref=local_bf, dst_ref=right_bf,
            send_sem=send_a.at[1], recv_sem=recv_a.at[1],
            device_id=(left,), device_id_type=pl.DeviceIdType.MESH)
        rdma_a_r.start()
        rdma_a_l.start()

        n_wt = n_per // WC
        for t in range(2):
            pltpu.make_async_copy(
                w_ref.at[:, pl.ds(t * WC, WC)], wf32.at[t],
                wc_sems.at[t]).start()

        def wconv(t, carry):
            s = t % 2
            pltpu.make_async_copy(
                w_ref.at[:, pl.ds(t * WC, WC)], wf32.at[s],
                wc_sems.at[s]).wait()
            w_bf[:, pl.ds(t * WC, WC)] = wf32[s].astype(jnp.bfloat16)

            @pl.when(t + 2 < n_wt)
            def _():
                pltpu.make_async_copy(
                    w_ref.at[:, pl.ds((t + 2) * WC, WC)], wf32.at[s],
                    wc_sems.at[s]).start()
            return carry

        lax.fori_loop(0, n_wt, wconv, 0)

        def compute_pair(ref_a, ra0, base_a, ref_b, rb0, base_b, nr):
            def emit(d, vals, base, col0):
                drain(d)
                y_vmem[d, pl.ds(0, nr), :] = jnp.maximum(vals, 0.0)
                cp = pltpu.make_async_copy(
                    y_vmem.at[d, pl.ds(0, nr), :],
                    out_ref.at[pl.ds(base, nr), pl.ds(col0, NBC)],
                    y_sems.at[d])
                cp.start()
                pending[d] = cp

            for t in range(n_tiles):
                ya = jnp.dot(ref_a[pl.ds(ra0, nr), :],
                             w_bf[:, pl.ds(t * NBC, NBC)],
                             preferred_element_type=jnp.float32)
                emit(0, ya, base_a, t * NBC)
                yb = jnp.dot(ref_b[pl.ds(rb0, nr), :],
                             w_bf[:, pl.ds(t * NBC, NBC)],
                             preferred_element_type=jnp.float32)
                emit(1, yb, base_b, t * NBC)

        compute_pair(local_bf, 0, my_pos * m_per,
                     local_bf, half, my_pos * m_per + half, half)

        rdma_a_r.wait_recv()
        fwd_r = []
        for q in range(2):
            r = pltpu.make_async_remote_copy(
                src_ref=left_bf.at[pl.ds(q * quart, quart), :],
                dst_ref=diag_bf.at[pl.ds(q * quart, quart), :],
                send_sem=send_b.at[q], recv_sem=recv_b.at[q],
                device_id=(right,), device_id_type=pl.DeviceIdType.MESH)
            r.start()
            fwd_r.append(r)
        rdma_a_l.wait_recv()
        fwd_l = []
        for q in range(2):
            r = pltpu.make_async_remote_copy(
                src_ref=right_bf.at[pl.ds(half + q * quart, quart), :],
                dst_ref=diag_bf.at[pl.ds(half + q * quart, quart), :],
                send_sem=send_b.at[2 + q], recv_sem=recv_b.at[2 + q],
                device_id=(left,), device_id_type=pl.DeviceIdType.MESH)
            r.start()
            fwd_l.append(r)

        compute_pair(left_bf, 0, left * m_per,
                     right_bf, 0, right * m_per, half)
        compute_pair(left_bf, half, left * m_per + half,
                     right_bf, half, right * m_per + half, half)
        rdma_a_r.wait_send()
        rdma_a_l.wait_send()

        diag = (my_pos + 2) % N_DEV
        fwd_r[0].wait_recv()
        fwd_l[0].wait_recv()
        compute_pair(diag_bf, 0, diag * m_per,
                     diag_bf, half, diag * m_per + half, quart)
        fwd_r[1].wait_recv()
        fwd_l[1].wait_recv()
        for r in fwd_r + fwd_l:
            r.wait_send()
        compute_pair(diag_bf, quart, diag * m_per + quart,
                     diag_bf, half + quart, diag * m_per + half + quart,
                     quart)

        drain(0)
        drain(1)

    return pl.pallas_call(
        body,
        out_shape=jax.ShapeDtypeStruct((N_DEV * m_per, n_per), jnp.float32),
        in_specs=[
            pl.BlockSpec(memory_space=pl.ANY),
            pl.BlockSpec(memory_space=pl.ANY),
        ],
        out_specs=pl.BlockSpec(memory_space=pl.ANY),
        scratch_shapes=[
            pltpu.VMEM((m_per, k), jnp.bfloat16),
            pltpu.VMEM((m_per, k), jnp.bfloat16),
            pltpu.VMEM((m_per, k), jnp.bfloat16),
            pltpu.VMEM((m_per, k), jnp.bfloat16),
            pltpu.VMEM((k, n_per), jnp.bfloat16),
            pltpu.VMEM((2, XR, k), jnp.float32),
            pltpu.VMEM((2, k, WC), jnp.float32),
            pltpu.VMEM((2, half, NBC), jnp.float32),
            pltpu.SemaphoreType.DMA((2,)),
            pltpu.SemaphoreType.DMA((2,)),
            pltpu.SemaphoreType.DMA((4,)),
            pltpu.SemaphoreType.DMA((4,)),
            pltpu.SemaphoreType.DMA((2,)),
            pltpu.SemaphoreType.DMA((2,)),
            pltpu.SemaphoreType.DMA((2,)),
        ],
        compiler_params=pltpu.CompilerParams(
            collective_id=0, vmem_limit_bytes=64 * 1024 * 1024),
    )(x, w_mat)


# device time: 180425 ns/iter; 1.8806x vs baseline; 1.0485x over previous
import jax
import jax.numpy as jnp
from jax import lax
from jax.experimental import pallas as pl
from jax.experimental.pallas import tpu as pltpu

N_DEV = 4
XR = 128
WC = 128
NBC = 512


def kernel(x, w_mat):
    m_per, k = x.shape
    k2, n_per = w_mat.shape
    assert k == k2
    half = m_per // 2
    quart = half // 2
    n_tiles = n_per // NBC

    def body(x_ref, w_ref, out_ref,
             local_bf, left_bf, right_bf, diag_bf, w_bf,
             xf32, wf32, y_vmem,
             send_a, recv_a, send_b, recv_b,
             xc_sems, wc_sems, y_sems):
        my_pos = lax.axis_index("i")
        left = (my_pos - 1) % N_DEV
        right = (my_pos + 1) % N_DEV

        barrier_sem = pltpu.get_barrier_semaphore()
        for nbr in [left, right]:
            pl.semaphore_signal(
                barrier_sem, inc=1,
                device_id=(nbr,), device_id_type=pl.DeviceIdType.MESH,
            )
        pl.semaphore_wait(barrier_sem, 2)

        pending = {}

        def drain(d):
            cp = pending.pop(d, None)
            if cp is not None:
                cp.wait()

        def xconv_range(t0, t1):
            for t in range(t0, t0 + 2):
                pltpu.make_async_copy(
                    x_ref.at[pl.ds(t * XR, XR), :], xf32.at[t % 2],
                    xc_sems.at[t % 2]).start()

            def step(t, carry):
                s = t % 2
                pltpu.make_async_copy(
                    x_ref.at[pl.ds(t * XR, XR), :], xf32.at[s],
                    xc_sems.at[s]).wait()
                local_bf[pl.ds(t * XR, XR), :] = \
                    xf32[s].astype(jnp.bfloat16)

                @pl.when(t + 2 < t1)
                def _():
                    pltpu.make_async_copy(
                        x_ref.at[pl.ds((t + 2) * XR, XR), :], xf32.at[s],
                        xc_sems.at[s]).start()
                return carry

            lax.fori_loop(t0, t1, step, 0)

        def half_send(r0, sem_i, tgt, dst_ref):
            r = pltpu.make_async_remote_copy(
                src_ref=local_bf.at[pl.ds(r0, half), :],
                dst_ref=dst_ref.at[pl.ds(r0, half), :],
                send_sem=send_a.at[sem_i], recv_sem=recv_a.at[sem_i],
                device_id=(tgt,), device_id_type=pl.DeviceIdType.MESH)
            r.start()
            return r

        xconv_range(0, half // XR)
        a1r = half_send(0, 0, right, left_bf)
        a1l = half_send(0, 1, left, right_bf)
        xconv_range(half // XR, m_per // XR)
        a2r = half_send(half, 2, right, left_bf)
        a2l = half_send(half, 3, left, right_bf)

        n_wt = n_per // WC
        for t in range(2):
            pltpu.make_async_copy(
                w_ref.at[:, pl.ds(t * WC, WC)], wf32.at[t],
                wc_sems.at[t]).start()

        def wconv(t, carry):
            s = t % 2
            pltpu.make_async_copy(
                w_ref.at[:, pl.ds(t * WC, WC)], wf32.at[s],
                wc_sems.at[s]).wait()
            w_bf[:, pl.ds(t * WC, WC)] = wf32[s].astype(jnp.bfloat16)

            @pl.when(t + 2 < n_wt)
            def _():
                pltpu.make_async_copy(
                    w_ref.at[:, pl.ds((t + 2) * WC, WC)], wf32.at[s],
                    wc_sems.at[s]).start()
            return carry

        lax.fori_loop(0, n_wt, wconv, 0)

        def compute_pair(ref_a, ra0, base_a, ref_b, rb0, base_b, nr):
            def emit(d, vals, base, col0):
                drain(d)
                y_vmem[d, pl.ds(0, nr), :] = jnp.maximum(vals, 0.0)
                cp = pltpu.make_async_copy(
                    y_vmem.at[d, pl.ds(0, nr), :],
                    out_ref.at[pl.ds(base, nr), pl.ds(col0, NBC)],
                    y_sems.at[d])
                cp.start()
                pending[d] = cp

            for t in range(n_tiles):
                ya = jnp.dot(ref_a[pl.ds(ra0, nr), :],
                             w_bf[:, pl.ds(t * NBC, NBC)],
                             preferred_element_type=jnp.float32)
                emit(0, ya, base_a, t * NBC)
                yb = jnp.dot(ref_b[pl.ds(rb0, nr), :],
                             w_bf[:, pl.ds(t * NBC, NBC)],
                             preferred_element_type=jnp.float32)
                emit(1, yb, base_b, t * NBC)

        compute_pair(local_bf, 0, my_pos * m_per,
                     local_bf, half, my_pos * m_per + half, half)

        a1r.wait_recv()
        fwd_r = []
        for q in range(2):
            r = pltpu.make_async_remote_copy(
                src_ref=left_bf.at[pl.ds(q * quart, quart), :],
                dst_ref=diag_bf.at[pl.ds(q * quart, quart), :],
                send_sem=send_b.at[q], recv_sem=recv_b.at[q],
                device_id=(right,), device_id_type=pl.DeviceIdType.MESH)
            r.start()
            fwd_r.append(r)
        a1l.wait_recv()
        compute_pair(left_bf, 0, left * m_per,
                     right_bf, 0, right * m_per, half)

        a2l.wait_recv()
        fwd_l = []
        for q in range(2):
            r = pltpu.make_async_remote_copy(
                src_ref=right_bf.at[pl.ds(half + q * quart, quart), :],
                dst_ref=diag_bf.at[pl.ds(half + q * quart, quart), :],
                send_sem=send_b.at[2 + q], recv_sem=recv_b.at[2 + q],
                device_id=(left,), device_id_type=pl.DeviceIdType.MESH)
            r.start()
            fwd_l.append(r)
        a2r.wait_recv()
        compute_pair(left_bf, half, left * m_per + half,
                     right_bf, half, right * m_per + half, half)
        for r in (a1r, a1l, a2r, a2l):
            r.wait_send()

        diag = (my_pos + 2) % N_DEV
        fwd_r[0].wait_recv()
        fwd_l[0].wait_recv()
        compute_pair(diag_bf, 0, diag * m_per,
                     diag_bf, half, diag * m_per + half, quart)
        fwd_r[1].wait_recv()
        fwd_l[1].wait_recv()
        for r in fwd_r + fwd_l:
            r.wait_send()
        compute_pair(diag_bf, quart, diag * m_per + quart,
                     diag_bf, half + quart, diag * m_per + half + quart,
                     quart)

        drain(0)
        drain(1)

    return pl.pallas_call(
        body,
        out_shape=jax.ShapeDtypeStruct((N_DEV * m_per, n_per), jnp.float32),
        in_specs=[
            pl.BlockSpec(memory_space=pl.ANY),
            pl.BlockSpec(memory_space=pl.ANY),
        ],
        out_specs=pl.BlockSpec(memory_space=pl.ANY),
        scratch_shapes=[
            pltpu.VMEM((m_per, k), jnp.bfloat16),
            pltpu.VMEM((m_per, k), jnp.bfloat16),
            pltpu.VMEM((m_per, k), jnp.bfloat16),
            pltpu.VMEM((m_per, k), jnp.bfloat16),
            pltpu.VMEM((k, n_per), jnp.bfloat16),
            pltpu.VMEM((2, XR, k), jnp.float32),
            pltpu.VMEM((2, k, WC), jnp.float32),
            pltpu.VMEM((2, half, NBC), jnp.float32),
            pltpu.SemaphoreType.DMA((4,)),
            pltpu.SemaphoreType.DMA((4,)),
            pltpu.SemaphoreType.DMA((4,)),
            pltpu.SemaphoreType.DMA((4,)),
            pltpu.SemaphoreType.DMA((2,)),
            pltpu.SemaphoreType.DMA((2,)),
            pltpu.SemaphoreType.DMA((2,)),
        ],
        compiler_params=pltpu.CompilerParams(
            collective_id=0, vmem_limit_bytes=64 * 1024 * 1024),
    )(x, w_mat)
